# baseline (speedup 1.0000x reference)
import functools

import jax
import jax.numpy as jnp
from jax import lax
from jax.experimental import pallas as pl
from jax.experimental.pallas import tpu as pltpu
from jax.experimental.pallas import tpu_sc as plsc


def _knn_body(x_ref, y_ref, idx_ref, w_ref, *, s_total):
    b = pl.program_id(0)
    x = x_ref[0]
    y = y_ref[0]
    x0, x1, x2 = x[:, 0:1], x[:, 1:2], x[:, 2:3]
    y0, y1, y2 = y[0:1, :], y[1:2, :], y[2:3, :]
    xb = x.astype(jnp.bfloat16).astype(jnp.float32)
    yb = y.astype(jnp.bfloat16).astype(jnp.float32)
    xb0, xb1, xb2 = xb[:, 0:1], xb[:, 1:2], xb[:, 2:3]
    yb0, yb1, yb2 = yb[0:1, :], yb[1:2, :], yb[2:3, :]
    acc = xb0 * yb0 + xb1 * yb1 + xb2 * yb2
    xsq = x0 * x0 + x1 * x1 + x2 * x2
    ysq = y0 * y0 + y1 * y1 + y2 * y2
    d = (-2.0 * acc + xsq) + ysq

    iota = lax.broadcasted_iota(jnp.int32, d.shape, 1)
    recips = []
    dcur = d
    for k in range(3):
        m = jnp.min(dcur, axis=1, keepdims=True)
        i = jnp.min(jnp.where(dcur == m, iota, s_total), axis=1,
                    keepdims=True)
        idx_ref[0, :, k:k + 1] = i + b * s_total
        recips.append(1.0 / (m + 1e-8))
        if k < 2:
            dcur = jnp.where(iota == i, jnp.inf, dcur)
    norm = recips[0] + recips[1] + recips[2]
    for k in range(3):
        w_ref[0, :, k:k + 1] = recips[k] / norm


def _knn(xyz1_t, xyz2, nb=512):
    bsz, n, _ = xyz1_t.shape
    _, _, s = xyz2.shape
    grid = (bsz, n // nb)
    out = pl.pallas_call(
        functools.partial(_knn_body, s_total=s),
        grid=grid,
        in_specs=[
            pl.BlockSpec((1, nb, 3), lambda b, i: (b, i, 0)),
            pl.BlockSpec((1, 3, s), lambda b, i: (b, 0, 0)),
        ],
        out_specs=[
            pl.BlockSpec((1, nb, 3), lambda b, i: (b, i, 0)),
            pl.BlockSpec((1, nb, 3), lambda b, i: (b, i, 0)),
        ],
        out_shape=[
            jax.ShapeDtypeStruct((bsz, n, 3), jnp.int32),
            jax.ShapeDtypeStruct((bsz, n, 3), jnp.float32),
        ],
    )(xyz1_t, xyz2)
    return out


_NC, _NS, _L = 2, 16, 16
_NW = _NC * _NS


def _sc_gather_interp(idxl, wgt, table, n, bsz, chunk=32):
    rows = bsz * n
    d2 = table.shape[1]
    rpw = rows // _NW
    nchunk = rpw // chunk
    assert rpw % chunk == 0 and nchunk % 2 == 0 and rpw <= n and n % rpw == 0
    c3 = chunk * 3

    mesh = plsc.VectorSubcoreMesh(core_axis_name="c", subcore_axis_name="s",
                                  num_cores=_NC, num_subcores=_NS)

    @functools.partial(
        pl.kernel,
        out_type=jax.ShapeDtypeStruct((rows, d2), jnp.float32),
        mesh=mesh,
        scratch_types=[
            pltpu.VMEM((rpw * 3,), jnp.int32),
            pltpu.VMEM((rpw * 3,), jnp.float32),
            pltpu.VMEM((c3, d2), jnp.float32),
            pltpu.VMEM((c3, d2), jnp.float32),
            pltpu.VMEM((chunk, d2), jnp.float32),
            pltpu.VMEM((chunk, d2), jnp.float32),
            pltpu.SemaphoreType.DMA,
            pltpu.SemaphoreType.DMA,
            pltpu.SemaphoreType.DMA,
            pltpu.SemaphoreType.DMA,
        ],
    )
    def gather_kernel(idx_hbm, w_hbm, tab_hbm, out_hbm,
                      ia, wa, rv0, rv1, ov0, ov1,
                      sg0, sg1, so0, so1):
        wid = lax.axis_index("s") * _NC + lax.axis_index("c")
        base = wid * rpw
        pltpu.sync_copy(idx_hbm.at[pl.ds(base * 3, rpw * 3)], ia)
        pltpu.sync_copy(w_hbm.at[pl.ds(base * 3, rpw * 3)], wa)

        bank0 = (rv0, ov0, sg0, so0)
        bank1 = (rv1, ov1, sg1, so1)

        def fire(c_idx, bank):
            rv, _, sg, _ = bank
            pltpu.async_copy(
                tab_hbm.at[ia.at[pl.ds(c_idx * c3, c3)]], rv, sg)

        def gwait(bank):
            rv, _, sg, _ = bank
            pltpu.make_async_copy(tab_hbm.at[pl.ds(0, c3)], rv, sg).wait()

        def owait(bank):
            _, ov, _, so = bank
            pltpu.make_async_copy(tab_hbm.at[pl.ds(0, chunk)], ov, so).wait()

        def compute_and_out(c_idx, bank):
            rv, ov, _, so = bank
            coff = c_idx * chunk

            def group_body(g, carry2):
                rbase = g * _L
                woff = (coff + rbase) * 3
                wv0 = wa[pl.ds(woff, _L)]
                wv1 = wa[pl.ds(woff + _L, _L)]
                wv2 = wa[pl.ds(woff + 2 * _L, _L)]
                wvs = (wv0, wv1, wv2)
                for t in range(_L):
                    r = rbase + t
                    fw = 3 * t
                    fr = 3 * r
                    s0_ = wvs[fw // _L][fw % _L]
                    s1_ = wvs[(fw + 1) // _L][(fw + 1) % _L]
                    s2_ = wvs[(fw + 2) // _L][(fw + 2) % _L]
                    for j in range(d2 // _L):
                        sl = pl.ds(j * _L, _L)
                        ov[r, sl] = (s0_ * rv[fr, sl] + s1_ * rv[fr + 1, sl]
                                     + s2_ * rv[fr + 2, sl])
                return carry2

            lax.fori_loop(0, chunk // _L, group_body, 0, unroll=False)
            pltpu.async_copy(ov, out_hbm.at[pl.ds(base + coff, chunk)], so)

        fire(0, bank0)
        fire(1, bank1)

        def body(ci2, carry):
            c = 2 * ci2
            gwait(bank0)
            compute_and_out(c, bank0)

            @pl.when(c + 2 < nchunk)
            def _():
                owait(bank0)
                fire(c + 2, bank0)

            gwait(bank1)
            compute_and_out(c + 1, bank1)

            @pl.when(c + 3 < nchunk)
            def _():
                owait(bank1)
                fire(c + 3, bank1)

            return carry

        lax.fori_loop(0, nchunk // 2, body, 0, unroll=False)
        owait(bank0)
        owait(bank1)

    return gather_kernel(idxl, wgt, table)


def _mlp1_body(p1_ref, it_ref, w0a_ref, w0b_ref, b0_ref, h1_ref, st_ref):
    b = pl.program_id(0)
    i = pl.program_id(1)
    x1 = p1_ref[0]
    xi = it_ref[0]
    h = lax.dot_general(x1, w0a_ref[...], (((0,), (0,)), ((), ())),
                        preferred_element_type=jnp.float32)
    h = h + lax.dot_general(xi, w0b_ref[...], (((1,), (0,)), ((), ())),
                            preferred_element_type=jnp.float32)
    h = h + b0_ref[...]
    h1_ref[0] = h

    @pl.when((b == 0) & (i == 0))
    def _():
        st_ref[...] = jnp.zeros_like(st_ref)

    st_ref[0:1, :] += jnp.sum(h, axis=0, keepdims=True)
    st_ref[1:2, :] += jnp.sum(h * h, axis=0, keepdims=True)


def _mlp1(points1, interp3, w0a, w0b, b0r, nb=512):
    bsz, d1, n = points1.shape
    h0 = w0a.shape[1]
    grid = (bsz, n // nb)
    return pl.pallas_call(
        _mlp1_body,
        grid=grid,
        in_specs=[
            pl.BlockSpec((1, d1, nb), lambda b, i: (b, 0, i)),
            pl.BlockSpec((1, nb, w0b.shape[0]), lambda b, i: (b, i, 0)),
            pl.BlockSpec(w0a.shape, lambda b, i: (0, 0)),
            pl.BlockSpec(w0b.shape, lambda b, i: (0, 0)),
            pl.BlockSpec((1, h0), lambda b, i: (0, 0)),
        ],
        out_specs=[
            pl.BlockSpec((1, nb, h0), lambda b, i: (b, i, 0)),
            pl.BlockSpec((2, h0), lambda b, i: (0, 0)),
        ],
        out_shape=[
            jax.ShapeDtypeStruct((bsz, n, h0), jnp.float32),
            jax.ShapeDtypeStruct((2, h0), jnp.float32),
        ],
    )(points1, interp3, w0a, w0b, b0r)


def _mlp2_body(h1_ref, st1_ref, sc0_ref, bi0_ref, w1_ref, b1_ref,
               h2t_ref, st2_ref, *, rows_total):
    b = pl.program_id(0)
    i = pl.program_id(1)
    mean = st1_ref[0:1, :] * (1.0 / rows_total)
    msq = st1_ref[1:2, :] * (1.0 / rows_total)
    var = msq - mean * mean
    a = sc0_ref[...] * lax.rsqrt(var + 1e-5)
    c = bi0_ref[...] - mean * a
    r = jnp.maximum(h1_ref[0] * a + c, 0.0)
    h2t = lax.dot_general(w1_ref[...], r, (((0,), (1,)), ((), ())),
                          preferred_element_type=jnp.float32)
    h2t = h2t + b1_ref[...]
    h2t_ref[0] = h2t

    @pl.when((b == 0) & (i == 0))
    def _():
        st2_ref[...] = jnp.zeros_like(st2_ref)

    st2_ref[:, 0:1] += jnp.sum(h2t, axis=1, keepdims=True)
    st2_ref[:, 1:2] += jnp.sum(h2t * h2t, axis=1, keepdims=True)


def _mlp2(h1, st1, sc0, bi0, w1m, b1c, nb=512):
    bsz, n, h0 = h1.shape
    h1ch = w1m.shape[1]
    grid = (bsz, n // nb)
    return pl.pallas_call(
        functools.partial(_mlp2_body, rows_total=bsz * n),
        grid=grid,
        in_specs=[
            pl.BlockSpec((1, nb, h0), lambda b, i: (b, i, 0)),
            pl.BlockSpec((2, h0), lambda b, i: (0, 0)),
            pl.BlockSpec((1, h0), lambda b, i: (0, 0)),
            pl.BlockSpec((1, h0), lambda b, i: (0, 0)),
            pl.BlockSpec(w1m.shape, lambda b, i: (0, 0)),
            pl.BlockSpec((h1ch, 1), lambda b, i: (0, 0)),
        ],
        out_specs=[
            pl.BlockSpec((1, h1ch, nb), lambda b, i: (b, 0, i)),
            pl.BlockSpec((h1ch, 2), lambda b, i: (0, 0)),
        ],
        out_shape=[
            jax.ShapeDtypeStruct((bsz, h1ch, n), jnp.float32),
            jax.ShapeDtypeStruct((h1ch, 2), jnp.float32),
        ],
    )(h1, st1, sc0, bi0, w1m, b1c)


def _bn2_body(h2t_ref, st2_ref, sc1_ref, bi1_ref, out_ref, *, rows_total):
    mean = st2_ref[:, 0:1] * (1.0 / rows_total)
    msq = st2_ref[:, 1:2] * (1.0 / rows_total)
    var = msq - mean * mean
    a = sc1_ref[...] * lax.rsqrt(var + 1e-5)
    c = bi1_ref[...] - mean * a
    out_ref[0] = jnp.maximum(h2t_ref[0] * a + c, 0.0)


def _bn2(h2t, st2, sc1, bi1, nb=2048):
    bsz, h1ch, n = h2t.shape
    grid = (bsz, n // nb)
    return pl.pallas_call(
        functools.partial(_bn2_body, rows_total=bsz * n),
        grid=grid,
        in_specs=[
            pl.BlockSpec((1, h1ch, nb), lambda b, i: (b, 0, i)),
            pl.BlockSpec((h1ch, 2), lambda b, i: (0, 0)),
            pl.BlockSpec((h1ch, 1), lambda b, i: (0, 0)),
            pl.BlockSpec((h1ch, 1), lambda b, i: (0, 0)),
        ],
        out_specs=pl.BlockSpec((1, h1ch, nb), lambda b, i: (b, 0, i)),
        out_shape=jax.ShapeDtypeStruct((bsz, h1ch, n), jnp.float32),
    )(h2t, st2, sc1, bi1)


def kernel(xyz1, xyz2, points1, points2, W0, b0, scale0, bias0,
           W1, b1, scale1, bias1):
    bsz, _, n = xyz1.shape
    _, _, s = xyz2.shape
    d1 = points1.shape[1]
    d2 = points2.shape[1]
    h0 = W0.shape[1]
    h1ch = W1.shape[1]

    xyz1_t = jnp.transpose(xyz1, (0, 2, 1))
    idxg, wgt = _knn(xyz1_t, xyz2)

    table = jnp.transpose(points2, (0, 2, 1)).reshape(bsz * s, d2)
    interp = _sc_gather_interp(idxg.reshape(-1), wgt.reshape(-1),
                               table, n, bsz)
    interp3 = interp.reshape(bsz, n, d2)

    w0a = W0[:d1]
    w0b = W0[d1:]
    h1, st1 = _mlp1(points1, interp3, w0a, w0b, b0.reshape(1, h0))
    h2t, st2 = _mlp2(h1, st1, scale0.reshape(1, h0), bias0.reshape(1, h0),
                     W1, b1.reshape(h1ch, 1))
    out = _bn2(h2t, st2, scale1.reshape(h1ch, 1), bias1.reshape(h1ch, 1))
    return out


# speedup vs baseline: 1.4299x; 1.4299x over previous
import functools

import jax
import jax.numpy as jnp
from jax import lax
from jax.experimental import pallas as pl
from jax.experimental.pallas import tpu as pltpu
from jax.experimental.pallas import tpu_sc as plsc


def _knn_body(x_ref, y_ref, idx_ref, w_ref, *, s_total):
    b = pl.program_id(0)
    x = x_ref[0]
    y = y_ref[0]
    x0, x1, x2 = x[0:1, :], x[1:2, :], x[2:3, :]
    y0, y1, y2 = y[:, 0:1], y[:, 1:2], y[:, 2:3]
    xb = x.astype(jnp.bfloat16).astype(jnp.float32)
    yb = y.astype(jnp.bfloat16).astype(jnp.float32)
    xb0, xb1, xb2 = xb[0:1, :], xb[1:2, :], xb[2:3, :]
    yb0, yb1, yb2 = yb[:, 0:1], yb[:, 1:2], yb[:, 2:3]
    acc = yb0 * xb0 + yb1 * xb1 + yb2 * xb2
    xsq = x0 * x0 + x1 * x1 + x2 * x2
    ysq = y0 * y0 + y1 * y1 + y2 * y2
    d = (-2.0 * acc + xsq) + ysq

    iota = lax.broadcasted_iota(jnp.int32, d.shape, 0)
    ubits = lax.bitcast_convert_type(d, jnp.int32)
    key = ubits ^ ((ubits >> 31) & jnp.int32(0x7FFFFFFF))
    pk = (key & jnp.int32(-1024)) | iota
    recips = []
    for k in range(3):
        mk = jnp.min(pk, axis=0, keepdims=True)
        i = mk & jnp.int32(1023)
        kb = (mk & jnp.int32(-1024)) | jnp.int32(512)
        ub = kb ^ ((kb >> 31) & jnp.int32(0x7FFFFFFF))
        m = lax.bitcast_convert_type(ub, jnp.float32)
        idx_ref[0, k:k + 1, :] = i + b * s_total
        recips.append(1.0 / (m + 1e-8))
        if k < 2:
            pk = jnp.where(pk == mk, jnp.int32(0x7FFFFFFF), pk)
    norm = recips[0] + recips[1] + recips[2]
    for k in range(3):
        w_ref[0, k:k + 1, :] = recips[k] / norm


def _knn(xyz1, xyz2_t, nb=512):
    bsz, _, n = xyz1.shape
    _, s, _ = xyz2_t.shape
    grid = (bsz, n // nb)
    out = pl.pallas_call(
        functools.partial(_knn_body, s_total=s),
        grid=grid,
        in_specs=[
            pl.BlockSpec((1, 3, nb), lambda b, i: (b, 0, i)),
            pl.BlockSpec((1, s, 3), lambda b, i: (b, 0, 0)),
        ],
        out_specs=[
            pl.BlockSpec((1, 3, nb), lambda b, i: (b, 0, i)),
            pl.BlockSpec((1, 3, nb), lambda b, i: (b, 0, i)),
        ],
        out_shape=[
            jax.ShapeDtypeStruct((bsz, 3, n), jnp.int32),
            jax.ShapeDtypeStruct((bsz, 3, n), jnp.float32),
        ],
    )(xyz1, xyz2_t)
    return out


_NC, _NS, _L = 2, 16, 16
_NW = _NC * _NS


def _sc_gather_interp(idxg, wgt, table, n, chunk=64):
    bsz = idxg.shape[0]
    rows = bsz * n
    d2 = table.shape[1]
    rpw = rows // _NW
    nchunk = rpw // chunk
    assert rpw % chunk == 0 and nchunk % 2 == 0 and rpw <= n and n % rpw == 0

    mesh = plsc.VectorSubcoreMesh(core_axis_name="c", subcore_axis_name="s",
                                  num_cores=_NC, num_subcores=_NS)

    @functools.partial(
        pl.kernel,
        out_type=jax.ShapeDtypeStruct((rows, d2), jnp.float32),
        mesh=mesh,
        scratch_types=[
            pltpu.VMEM((rpw,), jnp.int32),
            pltpu.VMEM((rpw,), jnp.int32),
            pltpu.VMEM((rpw,), jnp.int32),
            pltpu.VMEM((rpw,), jnp.float32),
            pltpu.VMEM((rpw,), jnp.float32),
            pltpu.VMEM((rpw,), jnp.float32),
            pltpu.VMEM((chunk, d2), jnp.float32),
            pltpu.VMEM((chunk, d2), jnp.float32),
            pltpu.VMEM((chunk, d2), jnp.float32),
            pltpu.VMEM((chunk, d2), jnp.float32),
            pltpu.VMEM((chunk, d2), jnp.float32),
            pltpu.VMEM((chunk, d2), jnp.float32),
            pltpu.SemaphoreType.DMA,
            pltpu.SemaphoreType.DMA,
            pltpu.SemaphoreType.DMA,
            pltpu.SemaphoreType.DMA,
        ],
    )
    def gather_kernel(idx_hbm, w_hbm, tab_hbm, out_hbm,
                      i0a, i1a, i2a, w0a, w1a, w2a,
                      r00, r01, r02, r10, r11, r12,
                      sg0, sg1, so0, so1):
        wid = lax.axis_index("s") * _NC + lax.axis_index("c")
        base = wid * rpw
        b = base // n
        nin = base % n
        for k, (ia, wa) in enumerate(((i0a, w0a), (i1a, w1a), (i2a, w2a))):
            foff = (b * 3 + k) * n + nin
            pltpu.sync_copy(idx_hbm.at[pl.ds(foff, rpw)], ia)
            pltpu.sync_copy(w_hbm.at[pl.ds(foff, rpw)], wa)

        bank0 = (r00, r01, r02, sg0, so0)
        bank1 = (r10, r11, r12, sg1, so1)

        def fire(c_idx, bank):
            r0, r1, r2, sg, _ = bank
            off = c_idx * chunk
            for ia, rb in ((i0a, r0), (i1a, r1), (i2a, r2)):
                pltpu.async_copy(tab_hbm.at[ia.at[pl.ds(off, chunk)]], rb, sg)

        def gwait(bank):
            r0, r1, r2, sg, _ = bank
            for rb in (r0, r1, r2):
                pltpu.make_async_copy(tab_hbm.at[pl.ds(0, chunk)], rb, sg).wait()

        def owait(bank):
            r0, _, _, _, so = bank
            pltpu.make_async_copy(tab_hbm.at[pl.ds(0, chunk)], r0, so).wait()

        def compute_and_out(c_idx, bank):
            r0, r1, r2, _, so = bank
            coff = c_idx * chunk

            def group_body(g, carry2):
                rbase = g * _L
                wv0 = w0a[pl.ds(coff + rbase, _L)]
                wv1 = w1a[pl.ds(coff + rbase, _L)]
                wv2 = w2a[pl.ds(coff + rbase, _L)]
                for t in range(_L):
                    r = rbase + t
                    s0_ = wv0[t]
                    s1_ = wv1[t]
                    s2_ = wv2[t]
                    for j in range(d2 // _L):
                        sl = pl.ds(j * _L, _L)
                        r0[r, sl] = (s0_ * r0[r, sl] + s1_ * r1[r, sl]
                                     + s2_ * r2[r, sl])
                return carry2

            lax.fori_loop(0, chunk // _L, group_body, 0, unroll=False)
            pltpu.async_copy(r0, out_hbm.at[pl.ds(base + coff, chunk)], so)

        fire(0, bank0)
        fire(1, bank1)

        def body(ci2, carry):
            c = 2 * ci2
            gwait(bank0)
            compute_and_out(c, bank0)

            @pl.when(c + 2 < nchunk)
            def _():
                owait(bank0)
                fire(c + 2, bank0)

            gwait(bank1)
            compute_and_out(c + 1, bank1)

            @pl.when(c + 3 < nchunk)
            def _():
                owait(bank1)
                fire(c + 3, bank1)

            return carry

        lax.fori_loop(0, nchunk // 2, body, 0, unroll=False)
        owait(bank0)
        owait(bank1)

    return gather_kernel(idxg.reshape(-1), wgt.reshape(-1), table)


def _mlp1_body(p1_ref, it_ref, w0a_ref, w0b_ref, b0_ref, h1_ref, st_ref):
    b = pl.program_id(0)
    i = pl.program_id(1)
    x1 = p1_ref[0]
    xi = it_ref[0]
    h = lax.dot_general(x1, w0a_ref[...], (((0,), (0,)), ((), ())),
                        preferred_element_type=jnp.float32)
    h = h + lax.dot_general(xi, w0b_ref[...], (((1,), (0,)), ((), ())),
                            preferred_element_type=jnp.float32)
    h = h + b0_ref[...]
    h1_ref[0] = h

    @pl.when((b == 0) & (i == 0))
    def _():
        st_ref[...] = jnp.zeros_like(st_ref)

    st_ref[0:1, :] += jnp.sum(h, axis=0, keepdims=True)
    st_ref[1:2, :] += jnp.sum(h * h, axis=0, keepdims=True)


def _mlp1(points1, interp3, w0a, w0b, b0r, nb=512):
    bsz, d1, n = points1.shape
    h0 = w0a.shape[1]
    grid = (bsz, n // nb)
    return pl.pallas_call(
        _mlp1_body,
        grid=grid,
        in_specs=[
            pl.BlockSpec((1, d1, nb), lambda b, i: (b, 0, i)),
            pl.BlockSpec((1, nb, w0b.shape[0]), lambda b, i: (b, i, 0)),
            pl.BlockSpec(w0a.shape, lambda b, i: (0, 0)),
            pl.BlockSpec(w0b.shape, lambda b, i: (0, 0)),
            pl.BlockSpec((1, h0), lambda b, i: (0, 0)),
        ],
        out_specs=[
            pl.BlockSpec((1, nb, h0), lambda b, i: (b, i, 0)),
            pl.BlockSpec((2, h0), lambda b, i: (0, 0)),
        ],
        out_shape=[
            jax.ShapeDtypeStruct((bsz, n, h0), jnp.float32),
            jax.ShapeDtypeStruct((2, h0), jnp.float32),
        ],
    )(points1, interp3, w0a, w0b, b0r)


def _mlp2_body(h1_ref, st1_ref, sc0_ref, bi0_ref, w1_ref, b1_ref,
               h2t_ref, st2_ref, *, rows_total):
    b = pl.program_id(0)
    i = pl.program_id(1)
    mean = st1_ref[0:1, :] * (1.0 / rows_total)
    msq = st1_ref[1:2, :] * (1.0 / rows_total)
    var = msq - mean * mean
    a = sc0_ref[...] * lax.rsqrt(var + 1e-5)
    c = bi0_ref[...] - mean * a
    r = jnp.maximum(h1_ref[0] * a + c, 0.0)
    h2t = lax.dot_general(w1_ref[...], r, (((0,), (1,)), ((), ())),
                          preferred_element_type=jnp.float32)
    h2t = h2t + b1_ref[...]
    h2t_ref[0] = h2t

    @pl.when((b == 0) & (i == 0))
    def _():
        st2_ref[...] = jnp.zeros_like(st2_ref)

    st2_ref[:, 0:1] += jnp.sum(h2t, axis=1, keepdims=True)
    st2_ref[:, 1:2] += jnp.sum(h2t * h2t, axis=1, keepdims=True)


def _mlp2(h1, st1, sc0, bi0, w1m, b1c, nb=512):
    bsz, n, h0 = h1.shape
    h1ch = w1m.shape[1]
    grid = (bsz, n // nb)
    return pl.pallas_call(
        functools.partial(_mlp2_body, rows_total=bsz * n),
        grid=grid,
        in_specs=[
            pl.BlockSpec((1, nb, h0), lambda b, i: (b, i, 0)),
            pl.BlockSpec((2, h0), lambda b, i: (0, 0)),
            pl.BlockSpec((1, h0), lambda b, i: (0, 0)),
            pl.BlockSpec((1, h0), lambda b, i: (0, 0)),
            pl.BlockSpec(w1m.shape, lambda b, i: (0, 0)),
            pl.BlockSpec((h1ch, 1), lambda b, i: (0, 0)),
        ],
        out_specs=[
            pl.BlockSpec((1, h1ch, nb), lambda b, i: (b, 0, i)),
            pl.BlockSpec((h1ch, 2), lambda b, i: (0, 0)),
        ],
        out_shape=[
            jax.ShapeDtypeStruct((bsz, h1ch, n), jnp.float32),
            jax.ShapeDtypeStruct((h1ch, 2), jnp.float32),
        ],
    )(h1, st1, sc0, bi0, w1m, b1c)


def _bn2_body(h2t_ref, st2_ref, sc1_ref, bi1_ref, out_ref, *, rows_total):
    mean = st2_ref[:, 0:1] * (1.0 / rows_total)
    msq = st2_ref[:, 1:2] * (1.0 / rows_total)
    var = msq - mean * mean
    a = sc1_ref[...] * lax.rsqrt(var + 1e-5)
    c = bi1_ref[...] - mean * a
    out_ref[0] = jnp.maximum(h2t_ref[0] * a + c, 0.0)


def _bn2(h2t, st2, sc1, bi1, nb=2048):
    bsz, h1ch, n = h2t.shape
    grid = (bsz, n // nb)
    return pl.pallas_call(
        functools.partial(_bn2_body, rows_total=bsz * n),
        grid=grid,
        in_specs=[
            pl.BlockSpec((1, h1ch, nb), lambda b, i: (b, 0, i)),
            pl.BlockSpec((h1ch, 2), lambda b, i: (0, 0)),
            pl.BlockSpec((h1ch, 1), lambda b, i: (0, 0)),
            pl.BlockSpec((h1ch, 1), lambda b, i: (0, 0)),
        ],
        out_specs=pl.BlockSpec((1, h1ch, nb), lambda b, i: (b, 0, i)),
        out_shape=jax.ShapeDtypeStruct((bsz, h1ch, n), jnp.float32),
    )(h2t, st2, sc1, bi1)


def kernel(xyz1, xyz2, points1, points2, W0, b0, scale0, bias0,
           W1, b1, scale1, bias1):
    bsz, _, n = xyz1.shape
    _, _, s = xyz2.shape
    d1 = points1.shape[1]
    d2 = points2.shape[1]
    h0 = W0.shape[1]
    h1ch = W1.shape[1]

    xyz2_t = jnp.transpose(xyz2, (0, 2, 1))
    idxg, wgt = _knn(xyz1, xyz2_t)

    table = jnp.transpose(points2, (0, 2, 1)).reshape(bsz * s, d2)
    interp = _sc_gather_interp(idxg, wgt, table, n)
    interp3 = interp.reshape(bsz, n, d2)

    w0a = W0[:d1]
    w0b = W0[d1:]
    h1, st1 = _mlp1(points1, interp3, w0a, w0b, b0.reshape(1, h0))
    h2t, st2 = _mlp2(h1, st1, scale0.reshape(1, h0), bias0.reshape(1, h0),
                     W1, b1.reshape(h1ch, 1))
    out = _bn2(h2t, st2, scale1.reshape(h1ch, 1), bias1.reshape(h1ch, 1))
    return out


# speedup vs baseline: 1.4384x; 1.0060x over previous
import functools

import jax
import jax.numpy as jnp
from jax import lax
from jax.experimental import pallas as pl
from jax.experimental.pallas import tpu as pltpu
from jax.experimental.pallas import tpu_sc as plsc


def _knn_body(x_ref, y_ref, idx_ref, w_ref, *, s_total):
    b = pl.program_id(0)
    x = x_ref[0]
    y = y_ref[0]
    x0, x1, x2 = x[0:1, :], x[1:2, :], x[2:3, :]
    y0, y1, y2 = y[:, 0:1], y[:, 1:2], y[:, 2:3]
    xb = x.astype(jnp.bfloat16).astype(jnp.float32)
    yb = y.astype(jnp.bfloat16).astype(jnp.float32)
    xb0, xb1, xb2 = xb[0:1, :], xb[1:2, :], xb[2:3, :]
    yb0, yb1, yb2 = yb[:, 0:1], yb[:, 1:2], yb[:, 2:3]
    acc = yb0 * xb0 + yb1 * xb1 + yb2 * xb2
    xsq = x0 * x0 + x1 * x1 + x2 * x2
    ysq = y0 * y0 + y1 * y1 + y2 * y2
    d = (-2.0 * acc + xsq) + ysq

    iota = lax.broadcasted_iota(jnp.int32, d.shape, 0)
    ubits = lax.bitcast_convert_type(d, jnp.int32)
    key = ubits ^ ((ubits >> 31) & jnp.int32(0x7FFFFFFF))
    pk = (key & jnp.int32(-1024)) | iota
    recips = []
    for k in range(3):
        mk = jnp.min(pk, axis=0, keepdims=True)
        i = mk & jnp.int32(1023)
        kb = (mk & jnp.int32(-1024)) | jnp.int32(512)
        ub = kb ^ ((kb >> 31) & jnp.int32(0x7FFFFFFF))
        m = lax.bitcast_convert_type(ub, jnp.float32)
        idx_ref[0, k:k + 1, :] = i + b * s_total
        recips.append(1.0 / (m + 1e-8))
        if k < 2:
            pk = jnp.where(pk == mk, jnp.int32(0x7FFFFFFF), pk)
    norm = recips[0] + recips[1] + recips[2]
    for k in range(3):
        w_ref[0, k:k + 1, :] = recips[k] / norm


def _knn(xyz1, xyz2_t, nb=512):
    bsz, _, n = xyz1.shape
    _, s, _ = xyz2_t.shape
    grid = (bsz, n // nb)
    out = pl.pallas_call(
        functools.partial(_knn_body, s_total=s),
        grid=grid,
        in_specs=[
            pl.BlockSpec((1, 3, nb), lambda b, i: (b, 0, i)),
            pl.BlockSpec((1, s, 3), lambda b, i: (b, 0, 0)),
        ],
        out_specs=[
            pl.BlockSpec((1, 3, nb), lambda b, i: (b, 0, i)),
            pl.BlockSpec((1, 3, nb), lambda b, i: (b, 0, i)),
        ],
        out_shape=[
            jax.ShapeDtypeStruct((bsz, 3, n), jnp.int32),
            jax.ShapeDtypeStruct((bsz, 3, n), jnp.float32),
        ],
    )(xyz1, xyz2_t)
    return out


_NC, _NS, _L = 2, 16, 16
_NW = _NC * _NS


def _sc_gather_interp(idxg, wgt, table, n, chunk=64):
    bsz = idxg.shape[0]
    rows = bsz * n
    d2 = table.shape[1]
    rpw = rows // _NW
    nchunk = rpw // chunk
    assert rpw % chunk == 0 and nchunk % 2 == 0 and rpw <= n and n % rpw == 0

    mesh = plsc.VectorSubcoreMesh(core_axis_name="c", subcore_axis_name="s",
                                  num_cores=_NC, num_subcores=_NS)

    @functools.partial(
        pl.kernel,
        out_type=jax.ShapeDtypeStruct((rows, d2), jnp.float32),
        mesh=mesh,
        scratch_types=[
            pltpu.VMEM((rpw,), jnp.int32),
            pltpu.VMEM((rpw,), jnp.int32),
            pltpu.VMEM((rpw,), jnp.int32),
            pltpu.VMEM((rpw,), jnp.float32),
            pltpu.VMEM((rpw,), jnp.float32),
            pltpu.VMEM((rpw,), jnp.float32),
            pltpu.VMEM((chunk, d2), jnp.float32),
            pltpu.VMEM((chunk, d2), jnp.float32),
            pltpu.VMEM((chunk, d2), jnp.float32),
            pltpu.VMEM((chunk, d2), jnp.float32),
            pltpu.VMEM((chunk, d2), jnp.float32),
            pltpu.VMEM((chunk, d2), jnp.float32),
            pltpu.SemaphoreType.DMA,
            pltpu.SemaphoreType.DMA,
            pltpu.SemaphoreType.DMA,
            pltpu.SemaphoreType.DMA,
        ],
    )
    def gather_kernel(idx_hbm, w_hbm, tab_hbm, out_hbm,
                      i0a, i1a, i2a, w0a, w1a, w2a,
                      r00, r01, r02, r10, r11, r12,
                      sg0, sg1, so0, so1):
        wid = lax.axis_index("s") * _NC + lax.axis_index("c")
        base = wid * rpw
        b = base // n
        nin = base % n
        for k, (ia, wa) in enumerate(((i0a, w0a), (i1a, w1a), (i2a, w2a))):
            foff = (b * 3 + k) * n + nin
            pltpu.sync_copy(idx_hbm.at[pl.ds(foff, rpw)], ia)
            pltpu.sync_copy(w_hbm.at[pl.ds(foff, rpw)], wa)

        bank0 = (r00, r01, r02, sg0, so0)
        bank1 = (r10, r11, r12, sg1, so1)

        def fire(c_idx, bank):
            r0, r1, r2, sg, _ = bank
            off = c_idx * chunk
            for ia, rb in ((i0a, r0), (i1a, r1), (i2a, r2)):
                pltpu.async_copy(tab_hbm.at[ia.at[pl.ds(off, chunk)]], rb, sg)

        def gwait(bank):
            r0, r1, r2, sg, _ = bank
            for rb in (r0, r1, r2):
                pltpu.make_async_copy(tab_hbm.at[pl.ds(0, chunk)], rb, sg).wait()

        def owait(bank):
            r0, _, _, _, so = bank
            pltpu.make_async_copy(tab_hbm.at[pl.ds(0, chunk)], r0, so).wait()

        def compute_and_out(c_idx, bank):
            r0, r1, r2, _, so = bank
            coff = c_idx * chunk

            def group_body(g, carry2):
                rbase = g * _L
                wv0 = w0a[pl.ds(coff + rbase, _L)]
                wv1 = w1a[pl.ds(coff + rbase, _L)]
                wv2 = w2a[pl.ds(coff + rbase, _L)]
                for t in range(_L):
                    r = rbase + t
                    s0_ = wv0[t]
                    s1_ = wv1[t]
                    s2_ = wv2[t]
                    for j in range(d2 // _L):
                        sl = pl.ds(j * _L, _L)
                        r0[r, sl] = (s0_ * r0[r, sl] + s1_ * r1[r, sl]
                                     + s2_ * r2[r, sl])
                return carry2

            lax.fori_loop(0, chunk // _L, group_body, 0, unroll=False)
            pltpu.async_copy(r0, out_hbm.at[pl.ds(base + coff, chunk)], so)

        fire(0, bank0)
        fire(1, bank1)

        def body(ci2, carry):
            c = 2 * ci2
            gwait(bank0)
            compute_and_out(c, bank0)

            @pl.when(c + 2 < nchunk)
            def _():
                owait(bank0)
                fire(c + 2, bank0)

            gwait(bank1)
            compute_and_out(c + 1, bank1)

            @pl.when(c + 3 < nchunk)
            def _():
                owait(bank1)
                fire(c + 3, bank1)

            return carry

        lax.fori_loop(0, nchunk // 2, body, 0, unroll=False)
        owait(bank0)
        owait(bank1)

    return gather_kernel(idxg.reshape(-1), wgt.reshape(-1), table)


def _mlp1_body(p1_ref, it_ref, w0a_ref, w0b_ref, b0_ref, h1_ref, st_ref):
    b = pl.program_id(0)
    i = pl.program_id(1)
    x1 = p1_ref[0].astype(jnp.bfloat16)
    xi = it_ref[0].astype(jnp.bfloat16)
    w0a = w0a_ref[...].astype(jnp.bfloat16)
    w0b = w0b_ref[...].astype(jnp.bfloat16)
    h = lax.dot_general(x1, w0a, (((0,), (0,)), ((), ())),
                        preferred_element_type=jnp.float32)
    h = h + lax.dot_general(xi, w0b, (((1,), (0,)), ((), ())),
                            preferred_element_type=jnp.float32)
    h = h + b0_ref[...]
    h1_ref[0] = h

    @pl.when((b == 0) & (i == 0))
    def _():
        st_ref[...] = jnp.zeros_like(st_ref)

    st_ref[0:1, :] += jnp.sum(h, axis=0, keepdims=True)
    st_ref[1:2, :] += jnp.sum(h * h, axis=0, keepdims=True)


def _mlp1(points1, interp3, w0a, w0b, b0r, nb=512):
    bsz, d1, n = points1.shape
    h0 = w0a.shape[1]
    grid = (bsz, n // nb)
    return pl.pallas_call(
        _mlp1_body,
        grid=grid,
        in_specs=[
            pl.BlockSpec((1, d1, nb), lambda b, i: (b, 0, i)),
            pl.BlockSpec((1, nb, w0b.shape[0]), lambda b, i: (b, i, 0)),
            pl.BlockSpec(w0a.shape, lambda b, i: (0, 0)),
            pl.BlockSpec(w0b.shape, lambda b, i: (0, 0)),
            pl.BlockSpec((1, h0), lambda b, i: (0, 0)),
        ],
        out_specs=[
            pl.BlockSpec((1, nb, h0), lambda b, i: (b, i, 0)),
            pl.BlockSpec((2, h0), lambda b, i: (0, 0)),
        ],
        out_shape=[
            jax.ShapeDtypeStruct((bsz, n, h0), jnp.float32),
            jax.ShapeDtypeStruct((2, h0), jnp.float32),
        ],
    )(points1, interp3, w0a, w0b, b0r)


def _mlp2_body(h1_ref, st1_ref, sc0_ref, bi0_ref, w1_ref, b1_ref,
               h2t_ref, st2_ref, *, rows_total):
    b = pl.program_id(0)
    i = pl.program_id(1)
    mean = st1_ref[0:1, :] * (1.0 / rows_total)
    msq = st1_ref[1:2, :] * (1.0 / rows_total)
    var = msq - mean * mean
    a = sc0_ref[...] * lax.rsqrt(var + 1e-5)
    c = bi0_ref[...] - mean * a
    r = jnp.maximum(h1_ref[0] * a + c, 0.0)
    h2t = lax.dot_general(w1_ref[...].astype(jnp.bfloat16),
                          r.astype(jnp.bfloat16), (((0,), (1,)), ((), ())),
                          preferred_element_type=jnp.float32)
    h2t = h2t + b1_ref[...]
    h2t_ref[0] = h2t

    @pl.when((b == 0) & (i == 0))
    def _():
        st2_ref[...] = jnp.zeros_like(st2_ref)

    st2_ref[:, 0:1] += jnp.sum(h2t, axis=1, keepdims=True)
    st2_ref[:, 1:2] += jnp.sum(h2t * h2t, axis=1, keepdims=True)


def _mlp2(h1, st1, sc0, bi0, w1m, b1c, nb=512):
    bsz, n, h0 = h1.shape
    h1ch = w1m.shape[1]
    grid = (bsz, n // nb)
    return pl.pallas_call(
        functools.partial(_mlp2_body, rows_total=bsz * n),
        grid=grid,
        in_specs=[
            pl.BlockSpec((1, nb, h0), lambda b, i: (b, i, 0)),
            pl.BlockSpec((2, h0), lambda b, i: (0, 0)),
            pl.BlockSpec((1, h0), lambda b, i: (0, 0)),
            pl.BlockSpec((1, h0), lambda b, i: (0, 0)),
            pl.BlockSpec(w1m.shape, lambda b, i: (0, 0)),
            pl.BlockSpec((h1ch, 1), lambda b, i: (0, 0)),
        ],
        out_specs=[
            pl.BlockSpec((1, h1ch, nb), lambda b, i: (b, 0, i)),
            pl.BlockSpec((h1ch, 2), lambda b, i: (0, 0)),
        ],
        out_shape=[
            jax.ShapeDtypeStruct((bsz, h1ch, n), jnp.float32),
            jax.ShapeDtypeStruct((h1ch, 2), jnp.float32),
        ],
    )(h1, st1, sc0, bi0, w1m, b1c)


def _bn2_body(h2t_ref, st2_ref, sc1_ref, bi1_ref, out_ref, *, rows_total):
    mean = st2_ref[:, 0:1] * (1.0 / rows_total)
    msq = st2_ref[:, 1:2] * (1.0 / rows_total)
    var = msq - mean * mean
    a = sc1_ref[...] * lax.rsqrt(var + 1e-5)
    c = bi1_ref[...] - mean * a
    out_ref[0] = jnp.maximum(h2t_ref[0] * a + c, 0.0)


def _bn2(h2t, st2, sc1, bi1, nb=2048):
    bsz, h1ch, n = h2t.shape
    grid = (bsz, n // nb)
    return pl.pallas_call(
        functools.partial(_bn2_body, rows_total=bsz * n),
        grid=grid,
        in_specs=[
            pl.BlockSpec((1, h1ch, nb), lambda b, i: (b, 0, i)),
            pl.BlockSpec((h1ch, 2), lambda b, i: (0, 0)),
            pl.BlockSpec((h1ch, 1), lambda b, i: (0, 0)),
            pl.BlockSpec((h1ch, 1), lambda b, i: (0, 0)),
        ],
        out_specs=pl.BlockSpec((1, h1ch, nb), lambda b, i: (b, 0, i)),
        out_shape=jax.ShapeDtypeStruct((bsz, h1ch, n), jnp.float32),
    )(h2t, st2, sc1, bi1)


def kernel(xyz1, xyz2, points1, points2, W0, b0, scale0, bias0,
           W1, b1, scale1, bias1):
    bsz, _, n = xyz1.shape
    _, _, s = xyz2.shape
    d1 = points1.shape[1]
    d2 = points2.shape[1]
    h0 = W0.shape[1]
    h1ch = W1.shape[1]

    xyz2_t = jnp.transpose(xyz2, (0, 2, 1))
    idxg, wgt = _knn(xyz1, xyz2_t)

    table = jnp.transpose(points2, (0, 2, 1)).reshape(bsz * s, d2)
    interp = _sc_gather_interp(idxg, wgt, table, n)
    interp3 = interp.reshape(bsz, n, d2)

    w0a = W0[:d1]
    w0b = W0[d1:]
    h1, st1 = _mlp1(points1, interp3, w0a, w0b, b0.reshape(1, h0))
    h2t, st2 = _mlp2(h1, st1, scale0.reshape(1, h0), bias0.reshape(1, h0),
                     W1, b1.reshape(h1ch, 1))
    out = _bn2(h2t, st2, scale1.reshape(h1ch, 1), bias1.reshape(h1ch, 1))
    return out
